# baseline (speedup 1.0000x reference)
import functools
import jax
import jax.numpy as jnp
from jax import lax
from jax.experimental import pallas as pl
from jax.experimental.pallas import tpu as pltpu
from jax.experimental.pallas import tpu_sc as plsc

N = 10000
NPAD = 10240
E = 320000
NC = 2
NS = 16
NW = NC * NS
LANES = 16

F32 = jnp.float32
I32 = jnp.int32


def _make_edge_pass(n, e, feat, heads, ch):
    epw = e // NW
    ce = 80
    chunks = epw // ce
    nsub = ce // LANES
    rpt = NPAD // NS
    logc = ch.bit_length() - 1
    nparts = feat // LANES

    mesh = plsc.VectorSubcoreMesh(core_axis_name="c", subcore_axis_name="s")

    @functools.partial(
        pl.kernel,
        out_type=[
            jax.ShapeDtypeStruct((NC, NPAD, feat), F32),
            jax.ShapeDtypeStruct((NC, NPAD, 16), F32),
        ],
        mesh=mesh,
        scratch_types=[
            pltpu.VMEM((2, ce), I32),
            pltpu.VMEM((2, ce), I32),
            pltpu.VMEM((2, ce, 16), F32),
            pltpu.VMEM((2, ce, 16), F32),
            pltpu.VMEM((2, ce, feat), F32),
            pltpu.VMEM((ce, 16), F32),
            pltpu.VMEM((ce, feat), F32),
            pltpu.VMEM_SHARED((NPAD, feat), F32),
            pltpu.VMEM_SHARED((NPAD, 16), F32),
            pltpu.SemaphoreType.DMA((2,)),
            pltpu.SemaphoreType.DMA((2,)),
            pltpu.SemaphoreType.DMA((2,)),
        ],
        compiler_params=pltpu.CompilerParams(use_tc_tiling_on_sc=False),
    )
    def edge_pass(al_s_hbm, al_d_hbm, h_hbm, src_hbm, dst_hbm,
                  zacc_hbm, zden_hbm,
                  acc_out, den_out,
                  src_v, dst_v, sa, da, hh, tbuf, msg, acc_s, den_s,
                  sem_a, sem_b, sem_h):
        cid = lax.axis_index("c")
        sid = lax.axis_index("s")
        wid = cid * NS + sid

        iota = lax.iota(I32, LANES)
        exp_idx = [
            lax.shift_right_logical(iota + LANES * j,
                                    jnp.full((LANES,), logc, I32))
            for j in range(nparts)
        ]

        gdn = lax.GatherDimensionNumbers(
            offset_dims=(), collapsed_slice_dims=(0,), start_index_map=(0,))

        def take(vec, idx):
            return lax.gather(
                vec, idx[:, None], dimension_numbers=gdn, slice_sizes=(1,),
                mode=lax.GatherScatterMode.PROMISE_IN_BOUNDS)

        def fire(c, b):
            pltpu.sync_copy(src_hbm.at[wid, c], src_v.at[b])
            pltpu.sync_copy(dst_hbm.at[wid, c], dst_v.at[b])
            pltpu.async_copy(al_s_hbm.at[src_v.at[b]], sa.at[b], sem_a.at[b])
            pltpu.async_copy(al_d_hbm.at[dst_v.at[b]], da.at[b], sem_b.at[b])
            pltpu.async_copy(h_hbm.at[src_v.at[b]], hh.at[b], sem_h.at[b])

        def drain(b):
            pltpu.make_async_copy(al_s_hbm.at[src_v.at[b]], sa.at[b],
                                  sem_a.at[b]).wait()
            pltpu.make_async_copy(al_d_hbm.at[dst_v.at[b]], da.at[b],
                                  sem_b.at[b]).wait()
            pltpu.make_async_copy(h_hbm.at[src_v.at[b]], hh.at[b],
                                  sem_h.at[b]).wait()

        row0 = sid * rpt
        pltpu.sync_copy(zacc_hbm, acc_s.at[pl.ds(row0, rpt)])
        pltpu.sync_copy(zden_hbm, den_s.at[pl.ds(row0, rpt)])
        plsc.subcore_barrier()

        fire(0, 0)
        fire(1, 1)

        def body(c, carry):
            b = lax.bitwise_and(c, 1)
            drain(b)
            for i in range(ce):
                ee = sa[b, i] + da[b, i]
                ee = jnp.maximum(ee, 0.2 * ee)
                tt = jnp.exp(ee)
                tbuf[i] = tt
                for j in range(nparts):
                    te = take(tt, exp_idx[j])
                    msg[i, pl.ds(LANES * j, LANES)] = (
                        te * hh[b, i, pl.ds(LANES * j, LANES)])
            for s in range(nsub):
                dst16 = dst_v[b, pl.ds(LANES * s, LANES)]
                pltpu.sync_copy(tbuf.at[pl.ds(LANES * s, LANES)],
                                den_s.at[dst16], add=True)
                pltpu.sync_copy(msg.at[pl.ds(LANES * s, LANES)],
                                acc_s.at[dst16], add=True)
            fire(jnp.minimum(c + 2, chunks - 1), b)
            return carry

        lax.fori_loop(0, chunks, body, 0)
        drain(0)
        drain(1)

        plsc.subcore_barrier()
        pltpu.sync_copy(acc_s.at[pl.ds(row0, rpt)],
                        acc_out.at[cid, pl.ds(row0, rpt)])
        pltpu.sync_copy(den_s.at[pl.ds(row0, rpt)],
                        den_out.at[cid, pl.ds(row0, rpt)])

    return edge_pass


_BLK = 1000
_GRID = N // _BLK


def _tc1_body(x_ref, w_ref, a_ref, aw_ref, h_ref, al_ref, ald_ref):
    h = jnp.dot(x_ref[...], w_ref[...], preferred_element_type=F32)
    h_ref[...] = h
    al_ref[...] = jnp.dot(h, a_ref[...], preferred_element_type=F32)
    ald_ref[...] = jnp.dot(h, aw_ref[...], preferred_element_type=F32)


def _tc2_body(acc0, acc1, den0, den1, b_ref, w_ref, a_ref, aw_ref, p_ref,
              h_ref, al_ref, ald_ref):
    acc = acc0[...] + acc1[...]
    den = jnp.dot(den0[...] + den1[...], p_ref[...],
                  preferred_element_type=F32)
    x1 = acc / jnp.maximum(den, 1e-16) + b_ref[...]
    act = jnp.where(x1 > 0, x1, jnp.exp(x1) - 1.0)
    h2 = jnp.dot(act, w_ref[...], preferred_element_type=F32)
    h_ref[...] = h2
    al_ref[...] = jnp.dot(h2, a_ref[...], preferred_element_type=F32)
    ald_ref[...] = jnp.dot(h2, aw_ref[...], preferred_element_type=F32)


def _tc3_body(acc0, acc1, den0, den1, b_ref, p_ref, out_ref):
    acc = acc0[...] + acc1[...]
    den = jnp.dot(den0[...] + den1[...], p_ref[...],
                  preferred_element_type=F32)
    out_ref[...] = acc / jnp.maximum(den, 1e-16) + b_ref[...]


def _row_spec(cols):
    return pl.BlockSpec((_BLK, cols), lambda i: (i, 0))


def _full_spec(rows, cols):
    return pl.BlockSpec((rows, cols), lambda i: (0, 0))


def kernel(x, edge_index, W1, a1_src, a1_dst, b1, W2, a2_src, a2_dst, b2):
    epw = E // NW
    ce = 80
    src = edge_index[0].reshape(NW, epw // ce, ce)
    dst = edge_index[1].reshape(NW, epw // ce, ce)

    def pack_a(a_s, a_d, din):
        h_, c_ = a_s.shape
        cols = []
        for k in range(16):
            if k < h_:
                col = jnp.zeros((din,), F32).at[k * c_:(k + 1) * c_].set(a_s[k])
            elif 8 <= k < 8 + h_:
                hh = k - 8
                col = jnp.zeros((din,), F32).at[hh * c_:(hh + 1) * c_].set(a_d[hh])
            else:
                col = jnp.zeros((din,), F32)
            cols.append(col)
        return jnp.stack(cols, axis=1)

    A1 = pack_a(a1_src, a1_dst, 64)
    A1w = pack_a(a1_dst, a1_src, 64)
    A2 = pack_a(a2_src, a2_dst, 128)
    A2w = pack_a(a2_dst, a2_src, 128)
    P1 = jnp.concatenate(
        [jnp.kron(jnp.eye(8, dtype=F32), jnp.ones((1, 8), F32)),
         jnp.zeros((8, 64), F32)], axis=0)
    P2 = jnp.concatenate(
        [jnp.ones((1, 128), F32), jnp.zeros((15, 128), F32)], axis=0)

    rpt = NPAD // NS
    z64 = jnp.zeros((rpt, 64), F32)
    z128 = jnp.zeros((rpt, 128), F32)
    z16 = jnp.zeros((rpt, 16), F32)

    h1, al1, al1d = pl.pallas_call(
        _tc1_body,
        grid=(_GRID,),
        in_specs=[_row_spec(128), _full_spec(128, 64), _full_spec(64, 16),
                  _full_spec(64, 16)],
        out_specs=[_row_spec(64), _row_spec(16), _row_spec(16)],
        out_shape=[jax.ShapeDtypeStruct((N, 64), F32),
                   jax.ShapeDtypeStruct((N, 16), F32),
                   jax.ShapeDtypeStruct((N, 16), F32)],
    )(x, W1, A1, A1w)

    acc1, den1 = _make_edge_pass(N, E, 64, 8, 8)(al1, al1d, h1, src, dst,
                                                 z64, z16)
    acc1 = acc1[:, :N]
    den1 = den1[:, :N]

    h2, al2, al2d = pl.pallas_call(
        _tc2_body,
        grid=(_GRID,),
        in_specs=[_row_spec(64), _row_spec(64), _row_spec(16), _row_spec(16),
                  _full_spec(1, 64), _full_spec(64, 128), _full_spec(128, 16),
                  _full_spec(128, 16), _full_spec(16, 64)],
        out_specs=[_row_spec(128), _row_spec(16), _row_spec(16)],
        out_shape=[jax.ShapeDtypeStruct((N, 128), F32),
                   jax.ShapeDtypeStruct((N, 16), F32),
                   jax.ShapeDtypeStruct((N, 16), F32)],
    )(acc1[0], acc1[1], den1[0], den1[1], b1.reshape(1, 64), W2, A2, A2w, P1)

    acc2, den2 = _make_edge_pass(N, E, 128, 1, 128)(al2, al2d, h2, src, dst,
                                                    z128, z16)
    acc2 = acc2[:, :N]
    den2 = den2[:, :N]

    out = pl.pallas_call(
        _tc3_body,
        grid=(_GRID,),
        in_specs=[_row_spec(128), _row_spec(128), _row_spec(16), _row_spec(16),
                  _full_spec(1, 128), _full_spec(16, 128)],
        out_specs=_row_spec(128),
        out_shape=jax.ShapeDtypeStruct((N, 128), F32),
    )(acc2[0], acc2[1], den2[0], den2[1], b2.reshape(1, 128), P2)

    return out


# speedup vs baseline: 2.3088x; 2.3088x over previous
import functools
import jax
import jax.numpy as jnp
from jax import lax
from jax.experimental import pallas as pl
from jax.experimental.pallas import tpu as pltpu
from jax.experimental.pallas import tpu_sc as plsc

N = 10000
NPAD = 10240
E = 320000
NC = 2
NS = 16
NW = NC * NS
LANES = 16

F32 = jnp.float32
I32 = jnp.int32


def _make_edge_pass(n, e, feat, heads, ch):
    epw = e // NW
    chunks = epw // LANES
    rpt = NPAD // NS
    logc = ch.bit_length() - 1
    nparts = feat // LANES

    mesh = plsc.VectorSubcoreMesh(core_axis_name="c", subcore_axis_name="s")

    @functools.partial(
        pl.kernel,
        out_type=[
            jax.ShapeDtypeStruct((NC, NPAD, feat), F32),
            jax.ShapeDtypeStruct((NC, NPAD, 16), F32),
        ],
        mesh=mesh,
        scratch_types=[
            pltpu.VMEM((chunks, LANES), I32),
            pltpu.VMEM((chunks, LANES), I32),
            pltpu.VMEM((LANES, 16), F32),
            pltpu.VMEM((LANES, 16), F32),
            pltpu.VMEM((LANES, 16), F32),
            pltpu.VMEM((LANES, 16), F32),
            pltpu.VMEM((LANES, feat), F32),
            pltpu.VMEM((LANES, feat), F32),
            pltpu.VMEM((LANES, 16), F32),
            pltpu.VMEM((LANES, feat), F32),
            pltpu.VMEM_SHARED((NPAD, feat), F32),
            pltpu.VMEM_SHARED((NPAD, 16), F32),
            pltpu.SemaphoreType.DMA((2,)),
            pltpu.SemaphoreType.DMA((2,)),
            pltpu.SemaphoreType.DMA((2,)),
        ],
        compiler_params=pltpu.CompilerParams(use_tc_tiling_on_sc=False),
    )
    def edge_pass(al_s_hbm, al_d_hbm, h_hbm, src_hbm, dst_hbm,
                  zacc_hbm, zden_hbm,
                  acc_out, den_out,
                  src_v, dst_v, sa0, sa1, da0, da1, hh0, hh1, tbuf, msg,
                  acc_s, den_s, sem_a, sem_b, sem_h):
        cid = lax.axis_index("c")
        sid = lax.axis_index("s")
        wid = cid * NS + sid
        bufs = ((sa0, da0, hh0), (sa1, da1, hh1))

        iota = lax.iota(I32, LANES)
        exp_idx = [
            lax.shift_right_logical(iota + LANES * j,
                                    jnp.full((LANES,), logc, I32))
            for j in range(nparts)
        ]

        gdn = lax.GatherDimensionNumbers(
            offset_dims=(), collapsed_slice_dims=(0,), start_index_map=(0,))

        def take(vec, idx):
            return lax.gather(
                vec, idx[:, None], dimension_numbers=gdn, slice_sizes=(1,),
                mode=lax.GatherScatterMode.PROMISE_IN_BOUNDS)

        def fire(c, b):
            sa, da, hh = bufs[b]
            src16 = src_v[c]
            pltpu.async_copy(al_s_hbm.at[src16], sa, sem_a.at[b])
            pltpu.async_copy(al_d_hbm.at[dst_v[c]], da, sem_b.at[b])
            pltpu.async_copy(h_hbm.at[src16], hh, sem_h.at[b])

        def drain(b):
            sa, da, hh = bufs[b]
            pltpu.make_async_copy(al_s_hbm.at[src_v[0]], sa,
                                  sem_a.at[b]).wait()
            pltpu.make_async_copy(al_d_hbm.at[dst_v[0]], da,
                                  sem_b.at[b]).wait()
            pltpu.make_async_copy(h_hbm.at[src_v[0]], hh,
                                  sem_h.at[b]).wait()

        def compute(c, b):
            sa, da, hh = bufs[b]
            for i in range(LANES):
                ee = sa[i] + da[i]
                ee = jnp.maximum(ee, 0.2 * ee)
                tt = jnp.exp(ee)
                tbuf[i] = tt
                for j in range(nparts):
                    te = take(tt, exp_idx[j])
                    msg[i, pl.ds(LANES * j, LANES)] = (
                        te * hh[i, pl.ds(LANES * j, LANES)])
            dst16 = dst_v[c]
            pltpu.sync_copy(tbuf, den_s.at[dst16], add=True)
            pltpu.sync_copy(msg, acc_s.at[dst16], add=True)

        row0 = sid * rpt
        pltpu.sync_copy(zacc_hbm, acc_s.at[pl.ds(row0, rpt)])
        pltpu.sync_copy(zden_hbm, den_s.at[pl.ds(row0, rpt)])
        pltpu.sync_copy(src_hbm.at[wid], src_v)
        pltpu.sync_copy(dst_hbm.at[wid], dst_v)
        plsc.subcore_barrier()

        last = chunks - 1
        fire(0, 0)
        fire(1, 1)

        def body(g2, carry):
            c0 = 2 * g2
            drain(0)
            compute(c0, 0)
            fire(jnp.minimum(c0 + 2, last), 0)
            c1 = c0 + 1
            drain(1)
            compute(c1, 1)
            fire(jnp.minimum(c1 + 2, last), 1)
            return carry

        lax.fori_loop(0, chunks // 2, body, 0)
        drain(0)
        compute(last, 0)
        drain(1)

        plsc.subcore_barrier()
        pltpu.sync_copy(acc_s.at[pl.ds(row0, rpt)],
                        acc_out.at[cid, pl.ds(row0, rpt)])
        pltpu.sync_copy(den_s.at[pl.ds(row0, rpt)],
                        den_out.at[cid, pl.ds(row0, rpt)])

    return edge_pass


_BLK = 1000
_GRID = N // _BLK


def _tc1_body(x_ref, w_ref, a_ref, aw_ref, h_ref, al_ref, ald_ref):
    h = jnp.dot(x_ref[...], w_ref[...], preferred_element_type=F32)
    h_ref[...] = h
    al_ref[...] = jnp.dot(h, a_ref[...], preferred_element_type=F32)
    ald_ref[...] = jnp.dot(h, aw_ref[...], preferred_element_type=F32)


def _tc2_body(acc0, acc1, den0, den1, b_ref, w_ref, a_ref, aw_ref, p_ref,
              h_ref, al_ref, ald_ref):
    acc = acc0[...] + acc1[...]
    den = jnp.dot(den0[...] + den1[...], p_ref[...],
                  preferred_element_type=F32)
    x1 = acc / jnp.maximum(den, 1e-16) + b_ref[...]
    act = jnp.where(x1 > 0, x1, jnp.exp(x1) - 1.0)
    h2 = jnp.dot(act, w_ref[...], preferred_element_type=F32)
    h_ref[...] = h2
    al_ref[...] = jnp.dot(h2, a_ref[...], preferred_element_type=F32)
    ald_ref[...] = jnp.dot(h2, aw_ref[...], preferred_element_type=F32)


def _tc3_body(acc0, acc1, den0, den1, b_ref, p_ref, out_ref):
    acc = acc0[...] + acc1[...]
    den = jnp.dot(den0[...] + den1[...], p_ref[...],
                  preferred_element_type=F32)
    out_ref[...] = acc / jnp.maximum(den, 1e-16) + b_ref[...]


def _row_spec(cols):
    return pl.BlockSpec((_BLK, cols), lambda i: (i, 0))


def _full_spec(rows, cols):
    return pl.BlockSpec((rows, cols), lambda i: (0, 0))


def kernel(x, edge_index, W1, a1_src, a1_dst, b1, W2, a2_src, a2_dst, b2):
    epw = E // NW
    src = edge_index[0].reshape(NW, epw // LANES, LANES)
    dst = edge_index[1].reshape(NW, epw // LANES, LANES)

    def pack_a(a_s, a_d, din):
        h_, c_ = a_s.shape
        cols = []
        for k in range(16):
            if k < h_:
                col = jnp.zeros((din,), F32).at[k * c_:(k + 1) * c_].set(a_s[k])
            elif 8 <= k < 8 + h_:
                hh = k - 8
                col = jnp.zeros((din,), F32).at[hh * c_:(hh + 1) * c_].set(a_d[hh])
            else:
                col = jnp.zeros((din,), F32)
            cols.append(col)
        return jnp.stack(cols, axis=1)

    A1 = pack_a(a1_src, a1_dst, 64)
    A1w = pack_a(a1_dst, a1_src, 64)
    A2 = pack_a(a2_src, a2_dst, 128)
    A2w = pack_a(a2_dst, a2_src, 128)
    P1 = jnp.concatenate(
        [jnp.kron(jnp.eye(8, dtype=F32), jnp.ones((1, 8), F32)),
         jnp.zeros((8, 64), F32)], axis=0)
    P2 = jnp.concatenate(
        [jnp.ones((1, 128), F32), jnp.zeros((15, 128), F32)], axis=0)

    rpt = NPAD // NS
    z64 = jnp.zeros((rpt, 64), F32)
    z128 = jnp.zeros((rpt, 128), F32)
    z16 = jnp.zeros((rpt, 16), F32)

    h1, al1, al1d = pl.pallas_call(
        _tc1_body,
        grid=(_GRID,),
        in_specs=[_row_spec(128), _full_spec(128, 64), _full_spec(64, 16),
                  _full_spec(64, 16)],
        out_specs=[_row_spec(64), _row_spec(16), _row_spec(16)],
        out_shape=[jax.ShapeDtypeStruct((N, 64), F32),
                   jax.ShapeDtypeStruct((N, 16), F32),
                   jax.ShapeDtypeStruct((N, 16), F32)],
    )(x, W1, A1, A1w)

    acc1, den1 = _make_edge_pass(N, E, 64, 8, 8)(al1, al1d, h1, src, dst,
                                                 z64, z16)
    acc1 = acc1[:, :N]
    den1 = den1[:, :N]

    h2, al2, al2d = pl.pallas_call(
        _tc2_body,
        grid=(_GRID,),
        in_specs=[_row_spec(64), _row_spec(64), _row_spec(16), _row_spec(16),
                  _full_spec(1, 64), _full_spec(64, 128), _full_spec(128, 16),
                  _full_spec(128, 16), _full_spec(16, 64)],
        out_specs=[_row_spec(128), _row_spec(16), _row_spec(16)],
        out_shape=[jax.ShapeDtypeStruct((N, 128), F32),
                   jax.ShapeDtypeStruct((N, 16), F32),
                   jax.ShapeDtypeStruct((N, 16), F32)],
    )(acc1[0], acc1[1], den1[0], den1[1], b1.reshape(1, 64), W2, A2, A2w, P1)

    acc2, den2 = _make_edge_pass(N, E, 128, 1, 128)(al2, al2d, h2, src, dst,
                                                    z128, z16)
    acc2 = acc2[:, :N]
    den2 = den2[:, :N]

    out = pl.pallas_call(
        _tc3_body,
        grid=(_GRID,),
        in_specs=[_row_spec(128), _row_spec(128), _row_spec(16), _row_spec(16),
                  _full_spec(1, 128), _full_spec(16, 128)],
        out_specs=_row_spec(128),
        out_shape=jax.ShapeDtypeStruct((N, 128), F32),
    )(acc2[0], acc2[1], den2[0], den2[1], b2.reshape(1, 128), P2)

    return out


# speedup vs baseline: 2.3922x; 1.0362x over previous
import functools
import jax
import jax.numpy as jnp
from jax import lax
from jax.experimental import pallas as pl
from jax.experimental.pallas import tpu as pltpu
from jax.experimental.pallas import tpu_sc as plsc

N = 10000
NPAD = 10240
E = 320000
NC = 2
NS = 16
NW = NC * NS
LANES = 16

F32 = jnp.float32
I32 = jnp.int32


def _make_edge_pass(feat, heads, ch):
    chunks = (E // NW) // LANES
    rpt = NPAD // NS
    logc = ch.bit_length() - 1
    nparts = feat // LANES

    mesh = plsc.VectorSubcoreMesh(core_axis_name="c", subcore_axis_name="s")

    @functools.partial(
        pl.kernel,
        out_type=[
            jax.ShapeDtypeStruct((NC, NPAD, feat), F32),
            jax.ShapeDtypeStruct((NC, NPAD, 16), F32),
        ],
        mesh=mesh,
        scratch_types=[
            pltpu.VMEM((chunks, LANES), I32),
            pltpu.VMEM((chunks, LANES), I32),
            pltpu.VMEM((LANES, 16), F32),
            pltpu.VMEM((LANES, 16), F32),
            pltpu.VMEM((LANES, 16), F32),
            pltpu.VMEM((LANES, 16), F32),
            pltpu.VMEM((LANES, feat), F32),
            pltpu.VMEM((LANES, feat), F32),
            pltpu.VMEM((LANES, 16), F32),
            pltpu.VMEM((LANES, feat), F32),
            pltpu.VMEM_SHARED((NPAD, feat), F32),
            pltpu.VMEM_SHARED((NPAD, 16), F32),
            pltpu.SemaphoreType.DMA((2,)),
            pltpu.SemaphoreType.DMA((2,)),
            pltpu.SemaphoreType.DMA((2,)),
        ],
        compiler_params=pltpu.CompilerParams(use_tc_tiling_on_sc=False),
    )
    def edge_pass(al_s_hbm, al_d_hbm, h_hbm, src_hbm, dst_hbm,
                  zacc_hbm, zden_hbm,
                  acc_out, den_out,
                  src_v, dst_v, sa0, sa1, da0, da1, hh0, hh1, tbuf, msg,
                  acc_s, den_s, sem_a, sem_b, sem_h):
        cid = lax.axis_index("c")
        sid = lax.axis_index("s")
        wid = cid * NS + sid
        bufs = ((sa0, da0, hh0), (sa1, da1, hh1))

        iota = lax.iota(I32, LANES)
        exp_idx = [
            lax.shift_right_logical(iota + LANES * j,
                                    jnp.full((LANES,), logc, I32))
            for j in range(nparts)
        ]

        gdn = lax.GatherDimensionNumbers(
            offset_dims=(), collapsed_slice_dims=(0,), start_index_map=(0,))

        def take(vec, idx):
            return lax.gather(
                vec, idx[:, None], dimension_numbers=gdn, slice_sizes=(1,),
                mode=lax.GatherScatterMode.PROMISE_IN_BOUNDS)

        def fire(c, b):
            sa, da, hh = bufs[b]
            src16 = src_v[c]
            pltpu.async_copy(al_s_hbm.at[src16], sa, sem_a.at[b])
            pltpu.async_copy(al_d_hbm.at[dst_v[c]], da, sem_b.at[b])
            pltpu.async_copy(h_hbm.at[src16], hh, sem_h.at[b])

        def drain(b):
            sa, da, hh = bufs[b]
            pltpu.make_async_copy(al_s_hbm.at[src_v[0]], sa,
                                  sem_a.at[b]).wait()
            pltpu.make_async_copy(al_d_hbm.at[dst_v[0]], da,
                                  sem_b.at[b]).wait()
            pltpu.make_async_copy(h_hbm.at[src_v[0]], hh,
                                  sem_h.at[b]).wait()

        def compute(c, b):
            sa, da, hh = bufs[b]
            for i in range(LANES):
                ee = sa[i] + da[i]
                ee = jnp.maximum(ee, 0.2 * ee)
                tt = jnp.exp(ee)
                tbuf[i] = tt
                if heads == 1:
                    te0 = take(tt, exp_idx[0])
                for j in range(nparts):
                    te = te0 if heads == 1 else take(tt, exp_idx[j])
                    msg[i, pl.ds(LANES * j, LANES)] = (
                        te * hh[i, pl.ds(LANES * j, LANES)])
            dst16 = dst_v[c]
            pltpu.sync_copy(tbuf, den_s.at[dst16], add=True)
            pltpu.sync_copy(msg, acc_s.at[dst16], add=True)

        row0 = sid * rpt
        pltpu.sync_copy(zacc_hbm, acc_s.at[pl.ds(row0, rpt)])
        pltpu.sync_copy(zden_hbm, den_s.at[pl.ds(row0, rpt)])
        pltpu.sync_copy(src_hbm.at[wid], src_v)
        pltpu.sync_copy(dst_hbm.at[wid], dst_v)
        plsc.subcore_barrier()

        last = chunks - 1
        fire(0, 0)
        fire(1, 1)

        def body(g2, carry):
            c0 = 2 * g2
            drain(0)
            compute(c0, 0)
            fire(jnp.minimum(c0 + 2, last), 0)
            c1 = c0 + 1
            drain(1)
            compute(c1, 1)
            fire(jnp.minimum(c1 + 2, last), 1)
            return carry

        lax.fori_loop(0, chunks // 2, body, 0)
        drain(0)
        compute(last, 0)
        drain(1)

        plsc.subcore_barrier()
        pltpu.sync_copy(acc_s.at[pl.ds(row0, rpt)],
                        acc_out.at[cid, pl.ds(row0, rpt)])
        pltpu.sync_copy(den_s.at[pl.ds(row0, rpt)],
                        den_out.at[cid, pl.ds(row0, rpt)])

    return edge_pass


_BLK = 1000
_GRID = N // _BLK


def _tc1_body(x_ref, w_ref, a_ref, aw_ref, h_ref, al_ref, ald_ref):
    h = jnp.dot(x_ref[...], w_ref[...], preferred_element_type=F32)
    h_ref[...] = h
    al_ref[...] = jnp.dot(h, a_ref[...], preferred_element_type=F32)
    ald_ref[...] = jnp.dot(h, aw_ref[...], preferred_element_type=F32)


def _tc2_body(acc_ref, den_ref, b_ref, w_ref, a_ref, aw_ref, p_ref,
              h_ref, al_ref, ald_ref):
    acc = acc_ref[0] + acc_ref[1]
    den = jnp.dot(den_ref[0] + den_ref[1], p_ref[...],
                  preferred_element_type=F32)
    x1 = acc / jnp.maximum(den, 1e-16) + b_ref[...]
    act = jnp.where(x1 > 0, x1, jnp.exp(x1) - 1.0)
    h2 = jnp.dot(act, w_ref[...], preferred_element_type=F32)
    h_ref[...] = h2
    al_ref[...] = jnp.dot(h2, a_ref[...], preferred_element_type=F32)
    ald_ref[...] = jnp.dot(h2, aw_ref[...], preferred_element_type=F32)


def _tc3_body(acc_ref, den_ref, b_ref, p_ref, out_ref):
    acc = acc_ref[0] + acc_ref[1]
    den = jnp.dot(den_ref[0] + den_ref[1], p_ref[...],
                  preferred_element_type=F32)
    out_ref[...] = acc / jnp.maximum(den, 1e-16) + b_ref[...]


def _row_spec(cols):
    return pl.BlockSpec((_BLK, cols), lambda i: (i, 0))


def _pair_spec(cols):
    return pl.BlockSpec((2, _BLK, cols), lambda i: (0, i, 0))


def _full_spec(rows, cols):
    return pl.BlockSpec((rows, cols), lambda i: (0, 0))


def kernel(x, edge_index, W1, a1_src, a1_dst, b1, W2, a2_src, a2_dst, b2):
    epw = E // NW
    src = edge_index[0].reshape(NW, epw // LANES, LANES)
    dst = edge_index[1].reshape(NW, epw // LANES, LANES)

    def pack_a(a_s, a_d, din):
        h_, c_ = a_s.shape
        cols = []
        for k in range(16):
            if k < h_:
                col = jnp.zeros((din,), F32).at[k * c_:(k + 1) * c_].set(a_s[k])
            elif 8 <= k < 8 + h_:
                hh = k - 8
                col = jnp.zeros((din,), F32).at[hh * c_:(hh + 1) * c_].set(a_d[hh])
            else:
                col = jnp.zeros((din,), F32)
            cols.append(col)
        return jnp.stack(cols, axis=1)

    A1 = pack_a(a1_src, a1_dst, 64)
    A1w = pack_a(a1_dst, a1_src, 64)
    A2 = pack_a(a2_src, a2_dst, 128)
    A2w = pack_a(a2_dst, a2_src, 128)
    P1 = jnp.concatenate(
        [jnp.kron(jnp.eye(8, dtype=F32), jnp.ones((1, 8), F32)),
         jnp.zeros((8, 64), F32)], axis=0)
    P2 = jnp.concatenate(
        [jnp.ones((1, 128), F32), jnp.zeros((15, 128), F32)], axis=0)

    rpt = NPAD // NS
    z64 = jnp.zeros((rpt, 64), F32)
    z128 = jnp.zeros((rpt, 128), F32)
    z16 = jnp.zeros((rpt, 16), F32)

    h1, al1, al1d = pl.pallas_call(
        _tc1_body,
        grid=(_GRID,),
        in_specs=[_row_spec(128), _full_spec(128, 64), _full_spec(64, 16),
                  _full_spec(64, 16)],
        out_specs=[_row_spec(64), _row_spec(16), _row_spec(16)],
        out_shape=[jax.ShapeDtypeStruct((N, 64), F32),
                   jax.ShapeDtypeStruct((N, 16), F32),
                   jax.ShapeDtypeStruct((N, 16), F32)],
    )(x, W1, A1, A1w)

    acc1, den1 = _make_edge_pass(64, 8, 8)(al1, al1d, h1, src, dst, z64, z16)

    h2, al2, al2d = pl.pallas_call(
        _tc2_body,
        grid=(_GRID,),
        in_specs=[_pair_spec(64), _pair_spec(16),
                  _full_spec(1, 64), _full_spec(64, 128), _full_spec(128, 16),
                  _full_spec(128, 16), _full_spec(16, 64)],
        out_specs=[_row_spec(128), _row_spec(16), _row_spec(16)],
        out_shape=[jax.ShapeDtypeStruct((N, 128), F32),
                   jax.ShapeDtypeStruct((N, 16), F32),
                   jax.ShapeDtypeStruct((N, 16), F32)],
    )(acc1, den1, b1.reshape(1, 64), W2, A2, A2w, P1)

    acc2, den2 = _make_edge_pass(128, 1, 128)(al2, al2d, h2, src, dst,
                                              z128, z16)

    out = pl.pallas_call(
        _tc3_body,
        grid=(_GRID,),
        in_specs=[_pair_spec(128), _pair_spec(16),
                  _full_spec(1, 128), _full_spec(16, 128)],
        out_specs=_row_spec(128),
        out_shape=jax.ShapeDtypeStruct((N, 128), F32),
    )(acc2, den2, b2.reshape(1, 128), P2)

    return out


# speedup vs baseline: 2.7653x; 1.1560x over previous
import functools
import jax
import jax.numpy as jnp
from jax import lax
from jax.experimental import pallas as pl
from jax.experimental.pallas import tpu as pltpu
from jax.experimental.pallas import tpu_sc as plsc

N = 10000
NPAD = 10240
E = 320000
NC = 2
NS = 16
NW = NC * NS
LANES = 16

F32 = jnp.float32
I32 = jnp.int32


def _make_edge_pass(feat, heads, ch):
    chunks = (E // NW) // LANES
    rpt = NPAD // NS
    logc = ch.bit_length() - 1
    nparts = feat // LANES

    mesh = plsc.VectorSubcoreMesh(core_axis_name="c", subcore_axis_name="s")

    @functools.partial(
        pl.kernel,
        out_type=[
            jax.ShapeDtypeStruct((NC, NPAD, feat), F32),
            jax.ShapeDtypeStruct((NC, NPAD, 16), F32),
        ],
        mesh=mesh,
        scratch_types=[
            pltpu.VMEM((chunks, LANES), I32),
            pltpu.VMEM((chunks, LANES), I32),
            pltpu.VMEM((LANES, 16), F32),
            pltpu.VMEM((LANES, 16), F32),
            pltpu.VMEM((LANES, 16), F32),
            pltpu.VMEM((LANES, 16), F32),
            pltpu.VMEM((LANES, feat), F32),
            pltpu.VMEM((LANES, feat), F32),
            pltpu.VMEM((LANES, 16), F32),
            pltpu.VMEM((LANES, 16), F32),
            pltpu.VMEM((LANES, feat), F32),
            pltpu.VMEM((LANES, feat), F32),
            pltpu.VMEM_SHARED((NPAD, feat), F32),
            pltpu.VMEM_SHARED((NPAD, 16), F32),
            pltpu.SemaphoreType.DMA((2,)),
            pltpu.SemaphoreType.DMA((2,)),
            pltpu.SemaphoreType.DMA((2,)),
            pltpu.SemaphoreType.DMA((2,)),
            pltpu.SemaphoreType.DMA((2,)),
        ],
        compiler_params=pltpu.CompilerParams(use_tc_tiling_on_sc=False),
    )
    def edge_pass(al_s_hbm, al_d_hbm, h_hbm, src_hbm, dst_hbm,
                  zacc_hbm, zden_hbm,
                  acc_out, den_out,
                  src_v, dst_v, sa0, sa1, da0, da1, hh0, hh1,
                  tb0, tb1, ms0, ms1,
                  acc_s, den_s, sem_a, sem_b, sem_h, sem_t, sem_m):
        cid = lax.axis_index("c")
        sid = lax.axis_index("s")
        wid = cid * NS + sid
        bufs = ((sa0, da0, hh0), (sa1, da1, hh1))
        sbufs = ((tb0, ms0), (tb1, ms1))

        iota = lax.iota(I32, LANES)
        exp_idx = [
            lax.shift_right_logical(iota + LANES * j,
                                    jnp.full((LANES,), logc, I32))
            for j in range(nparts)
        ]

        gdn = lax.GatherDimensionNumbers(
            offset_dims=(), collapsed_slice_dims=(0,), start_index_map=(0,))

        def take(vec, idx):
            return lax.gather(
                vec, idx[:, None], dimension_numbers=gdn, slice_sizes=(1,),
                mode=lax.GatherScatterMode.PROMISE_IN_BOUNDS)

        def fire(c, b):
            sa, da, hh = bufs[b]
            src16 = src_v[c]
            pltpu.async_copy(al_s_hbm.at[src16], sa, sem_a.at[b])
            pltpu.async_copy(al_d_hbm.at[dst_v[c]], da, sem_b.at[b])
            pltpu.async_copy(h_hbm.at[src16], hh, sem_h.at[b])

        def drain(b):
            sa, da, hh = bufs[b]
            pltpu.make_async_copy(al_s_hbm.at[src_v[0]], sa,
                                  sem_a.at[b]).wait()
            pltpu.make_async_copy(al_d_hbm.at[dst_v[0]], da,
                                  sem_b.at[b]).wait()
            pltpu.make_async_copy(h_hbm.at[src_v[0]], hh,
                                  sem_h.at[b]).wait()

        def compute(b):
            sa, da, hh = bufs[b]
            tbuf, msg = sbufs[b]
            for i in range(LANES):
                ee = sa[i] + da[i]
                ee = jnp.maximum(ee, 0.2 * ee)
                tt = jnp.exp(ee)
                tbuf[i] = tt
                if heads == 1:
                    te0 = take(tt, exp_idx[0])
                for j in range(nparts):
                    te = te0 if heads == 1 else take(tt, exp_idx[j])
                    msg[i, pl.ds(LANES * j, LANES)] = (
                        te * hh[i, pl.ds(LANES * j, LANES)])

        def fire_scatter(c, b):
            tbuf, msg = sbufs[b]
            dst16 = dst_v[c]
            pltpu.async_copy(tbuf, den_s.at[dst16], sem_t.at[b], add=True)
            pltpu.async_copy(msg, acc_s.at[dst16], sem_m.at[b], add=True)

        def wait_scatter(b):
            tbuf, msg = sbufs[b]
            pltpu.make_async_copy(tbuf, den_s.at[dst_v[0]],
                                  sem_t.at[b]).wait()
            pltpu.make_async_copy(msg, acc_s.at[dst_v[0]],
                                  sem_m.at[b]).wait()

        row0 = sid * rpt
        pltpu.sync_copy(zacc_hbm, acc_s.at[pl.ds(row0, rpt)])
        pltpu.sync_copy(zden_hbm, den_s.at[pl.ds(row0, rpt)])
        pltpu.sync_copy(src_hbm.at[wid], src_v)
        pltpu.sync_copy(dst_hbm.at[wid], dst_v)
        plsc.subcore_barrier()

        zv = jnp.zeros((LANES,), F32)
        for b in range(2):
            tbuf, msg = sbufs[b]
            for i in range(LANES):
                tbuf[i] = zv
                for j in range(nparts):
                    msg[i, pl.ds(LANES * j, LANES)] = zv
            fire_scatter(0, b)

        last = chunks - 1
        fire(0, 0)
        fire(1, 1)

        def body(g2, carry):
            c0 = 2 * g2
            drain(0)
            wait_scatter(0)
            compute(0)
            fire_scatter(c0, 0)
            fire(jnp.minimum(c0 + 2, last), 0)
            c1 = c0 + 1
            drain(1)
            wait_scatter(1)
            compute(1)
            fire_scatter(c1, 1)
            fire(jnp.minimum(c1 + 2, last), 1)
            return carry

        lax.fori_loop(0, chunks // 2, body, 0)
        drain(0)
        wait_scatter(0)
        compute(0)
        fire_scatter(last, 0)
        wait_scatter(0)
        wait_scatter(1)
        drain(1)

        plsc.subcore_barrier()
        pltpu.sync_copy(acc_s.at[pl.ds(row0, rpt)],
                        acc_out.at[cid, pl.ds(row0, rpt)])
        pltpu.sync_copy(den_s.at[pl.ds(row0, rpt)],
                        den_out.at[cid, pl.ds(row0, rpt)])

    return edge_pass


_BLK = 1000
_GRID = N // _BLK


def _tc1_body(x_ref, w_ref, a_ref, aw_ref, h_ref, al_ref, ald_ref):
    h = jnp.dot(x_ref[...], w_ref[...], preferred_element_type=F32)
    h_ref[...] = h
    al_ref[...] = jnp.dot(h, a_ref[...], preferred_element_type=F32)
    ald_ref[...] = jnp.dot(h, aw_ref[...], preferred_element_type=F32)


def _tc2_body(acc_ref, den_ref, b_ref, w_ref, a_ref, aw_ref, p_ref,
              h_ref, al_ref, ald_ref):
    acc = acc_ref[0] + acc_ref[1]
    den = jnp.dot(den_ref[0] + den_ref[1], p_ref[...],
                  preferred_element_type=F32)
    x1 = acc / jnp.maximum(den, 1e-16) + b_ref[...]
    act = jnp.where(x1 > 0, x1, jnp.exp(x1) - 1.0)
    h2 = jnp.dot(act, w_ref[...], preferred_element_type=F32)
    h_ref[...] = h2
    al_ref[...] = jnp.dot(h2, a_ref[...], preferred_element_type=F32)
    ald_ref[...] = jnp.dot(h2, aw_ref[...], preferred_element_type=F32)


def _tc3_body(acc_ref, den_ref, b_ref, p_ref, out_ref):
    acc = acc_ref[0] + acc_ref[1]
    den = jnp.dot(den_ref[0] + den_ref[1], p_ref[...],
                  preferred_element_type=F32)
    out_ref[...] = acc / jnp.maximum(den, 1e-16) + b_ref[...]


def _row_spec(cols):
    return pl.BlockSpec((_BLK, cols), lambda i: (i, 0))


def _pair_spec(cols):
    return pl.BlockSpec((2, _BLK, cols), lambda i: (0, i, 0))


def _full_spec(rows, cols):
    return pl.BlockSpec((rows, cols), lambda i: (0, 0))


def kernel(x, edge_index, W1, a1_src, a1_dst, b1, W2, a2_src, a2_dst, b2):
    epw = E // NW
    src = edge_index[0].reshape(NW, epw // LANES, LANES)
    dst = edge_index[1].reshape(NW, epw // LANES, LANES)

    def pack_a(a_s, a_d, din):
        h_, c_ = a_s.shape
        cols = []
        for k in range(16):
            if k < h_:
                col = jnp.zeros((din,), F32).at[k * c_:(k + 1) * c_].set(a_s[k])
            elif 8 <= k < 8 + h_:
                hh = k - 8
                col = jnp.zeros((din,), F32).at[hh * c_:(hh + 1) * c_].set(a_d[hh])
            else:
                col = jnp.zeros((din,), F32)
            cols.append(col)
        return jnp.stack(cols, axis=1)

    A1 = pack_a(a1_src, a1_dst, 64)
    A1w = pack_a(a1_dst, a1_src, 64)
    A2 = pack_a(a2_src, a2_dst, 128)
    A2w = pack_a(a2_dst, a2_src, 128)
    P1 = jnp.concatenate(
        [jnp.kron(jnp.eye(8, dtype=F32), jnp.ones((1, 8), F32)),
         jnp.zeros((8, 64), F32)], axis=0)
    P2 = jnp.concatenate(
        [jnp.ones((1, 128), F32), jnp.zeros((15, 128), F32)], axis=0)

    rpt = NPAD // NS
    z64 = jnp.zeros((rpt, 64), F32)
    z128 = jnp.zeros((rpt, 128), F32)
    z16 = jnp.zeros((rpt, 16), F32)

    h1, al1, al1d = pl.pallas_call(
        _tc1_body,
        grid=(_GRID,),
        in_specs=[_row_spec(128), _full_spec(128, 64), _full_spec(64, 16),
                  _full_spec(64, 16)],
        out_specs=[_row_spec(64), _row_spec(16), _row_spec(16)],
        out_shape=[jax.ShapeDtypeStruct((N, 64), F32),
                   jax.ShapeDtypeStruct((N, 16), F32),
                   jax.ShapeDtypeStruct((N, 16), F32)],
    )(x, W1, A1, A1w)

    acc1, den1 = _make_edge_pass(64, 8, 8)(al1, al1d, h1, src, dst, z64, z16)

    h2, al2, al2d = pl.pallas_call(
        _tc2_body,
        grid=(_GRID,),
        in_specs=[_pair_spec(64), _pair_spec(16),
                  _full_spec(1, 64), _full_spec(64, 128), _full_spec(128, 16),
                  _full_spec(128, 16), _full_spec(16, 64)],
        out_specs=[_row_spec(128), _row_spec(16), _row_spec(16)],
        out_shape=[jax.ShapeDtypeStruct((N, 128), F32),
                   jax.ShapeDtypeStruct((N, 16), F32),
                   jax.ShapeDtypeStruct((N, 16), F32)],
    )(acc1, den1, b1.reshape(1, 64), W2, A2, A2w, P1)

    acc2, den2 = _make_edge_pass(128, 1, 128)(al2, al2d, h2, src, dst,
                                              z128, z16)

    out = pl.pallas_call(
        _tc3_body,
        grid=(_GRID,),
        in_specs=[_pair_spec(128), _pair_spec(16),
                  _full_spec(1, 128), _full_spec(16, 128)],
        out_specs=_row_spec(128),
        out_shape=jax.ShapeDtypeStruct((N, 128), F32),
    )(acc2, den2, b2.reshape(1, 128), P2)

    return out


# speedup vs baseline: 3.4187x; 1.2363x over previous
import functools
import jax
import jax.numpy as jnp
from jax import lax
from jax.experimental import pallas as pl
from jax.experimental.pallas import tpu as pltpu
from jax.experimental.pallas import tpu_sc as plsc

N = 10000
NPAD = 10240
E = 320000
NC = 2
NS = 16
NW = NC * NS
LANES = 16

F32 = jnp.float32
I32 = jnp.int32


def _make_edge_pass(feat, heads, ch, ce):
    chunks = (E // NW) // ce
    nsub = ce // LANES
    rpt = NPAD // NS
    logc = ch.bit_length() - 1
    nparts = feat // LANES

    mesh = plsc.VectorSubcoreMesh(core_axis_name="c", subcore_axis_name="s")

    @functools.partial(
        pl.kernel,
        out_type=[
            jax.ShapeDtypeStruct((NC, NPAD, feat), F32),
            jax.ShapeDtypeStruct((NC, NPAD, 16), F32),
        ],
        mesh=mesh,
        scratch_types=[
            pltpu.VMEM((chunks, ce), I32),
            pltpu.VMEM((chunks, ce), I32),
            pltpu.VMEM((2, ce, 16), F32),
            pltpu.VMEM((2, ce, 16), F32),
            pltpu.VMEM((2, ce, feat), F32),
            pltpu.VMEM((2, ce, 16), F32),
            pltpu.VMEM((2, ce, feat), F32),
            pltpu.VMEM_SHARED((NPAD, feat), F32),
            pltpu.VMEM_SHARED((NPAD, 16), F32),
            pltpu.SemaphoreType.DMA((2,)),
            pltpu.SemaphoreType.DMA((2,)),
            pltpu.SemaphoreType.DMA((2,)),
            pltpu.SemaphoreType.DMA((2,)),
            pltpu.SemaphoreType.DMA((2,)),
        ],
        compiler_params=pltpu.CompilerParams(use_tc_tiling_on_sc=False),
    )
    def edge_pass(al_s_hbm, al_d_hbm, h_hbm, src_hbm, dst_hbm,
                  zacc_hbm, zden_hbm,
                  acc_out, den_out,
                  src_v, dst_v, sab, dab, hhb, tbb, msb,
                  acc_s, den_s, sem_a, sem_b, sem_h, sem_t, sem_m):
        cid = lax.axis_index("c")
        sid = lax.axis_index("s")
        wid = cid * NS + sid
        bufs = tuple((sab.at[b], dab.at[b], hhb.at[b]) for b in range(2))
        sbufs = tuple((tbb.at[b], msb.at[b]) for b in range(2))

        iota = lax.iota(I32, LANES)
        exp_idx = [
            lax.shift_right_logical(iota + LANES * j,
                                    jnp.full((LANES,), logc, I32))
            for j in range(nparts)
        ]

        gdn = lax.GatherDimensionNumbers(
            offset_dims=(), collapsed_slice_dims=(0,), start_index_map=(0,))

        def take(vec, idx):
            return lax.gather(
                vec, idx[:, None], dimension_numbers=gdn, slice_sizes=(1,),
                mode=lax.GatherScatterMode.PROMISE_IN_BOUNDS)

        def fire(c, b):
            sa, da, hh = bufs[b]
            pltpu.async_copy(al_s_hbm.at[src_v.at[c]], sa, sem_a.at[b])
            pltpu.async_copy(al_d_hbm.at[dst_v.at[c]], da, sem_b.at[b])
            pltpu.async_copy(h_hbm.at[src_v.at[c]], hh, sem_h.at[b])

        def drain(b):
            sa, da, hh = bufs[b]
            pltpu.make_async_copy(al_s_hbm.at[src_v.at[0]], sa,
                                  sem_a.at[b]).wait()
            pltpu.make_async_copy(al_d_hbm.at[dst_v.at[0]], da,
                                  sem_b.at[b]).wait()
            pltpu.make_async_copy(h_hbm.at[src_v.at[0]], hh,
                                  sem_h.at[b]).wait()

        def compute(b):
            sa, da, hh = bufs[b]
            tbuf, msg = sbufs[b]
            for i in range(ce):
                ee = sa[i] + da[i]
                ee = jnp.maximum(ee, 0.2 * ee)
                tt = jnp.exp(ee)
                tbuf[i] = tt
                if heads == 1:
                    te0 = take(tt, exp_idx[0])
                for j in range(nparts):
                    te = te0 if heads == 1 else take(tt, exp_idx[j])
                    msg[i, pl.ds(LANES * j, LANES)] = (
                        te * hh[i, pl.ds(LANES * j, LANES)])

        def fire_scatter(c, b):
            tbuf, msg = sbufs[b]
            for s in range(nsub):
                dst16 = dst_v[c, pl.ds(LANES * s, LANES)]
                pltpu.async_copy(tbuf.at[pl.ds(LANES * s, LANES)],
                                 den_s.at[dst16], sem_t.at[b], add=True)
                pltpu.async_copy(msg.at[pl.ds(LANES * s, LANES)],
                                 acc_s.at[dst16], sem_m.at[b], add=True)

        def wait_scatter(b):
            tbuf, msg = sbufs[b]
            for s in range(nsub):
                dst16 = dst_v[0, pl.ds(LANES * s, LANES)]
                pltpu.make_async_copy(tbuf.at[pl.ds(LANES * s, LANES)],
                                      den_s.at[dst16], sem_t.at[b]).wait()
                pltpu.make_async_copy(msg.at[pl.ds(LANES * s, LANES)],
                                      acc_s.at[dst16], sem_m.at[b]).wait()

        row0 = sid * rpt
        pltpu.sync_copy(zacc_hbm, acc_s.at[pl.ds(row0, rpt)])
        pltpu.sync_copy(zden_hbm, den_s.at[pl.ds(row0, rpt)])
        pltpu.sync_copy(src_hbm.at[wid], src_v)
        pltpu.sync_copy(dst_hbm.at[wid], dst_v)
        plsc.subcore_barrier()

        for b in range(2):
            tbuf, msg = sbufs[b]
            pltpu.sync_copy(zden_hbm.at[pl.ds(0, ce)], tbuf)
            pltpu.sync_copy(zacc_hbm.at[pl.ds(0, ce)], msg)
            fire_scatter(0, b)

        last = chunks - 1
        fire(0, 0)
        fire(1, 1)

        def body(g2, carry):
            c0 = 2 * g2
            drain(0)
            wait_scatter(0)
            compute(0)
            fire_scatter(c0, 0)
            fire(jnp.minimum(c0 + 2, last), 0)
            c1 = c0 + 1
            drain(1)
            wait_scatter(1)
            compute(1)
            fire_scatter(c1, 1)
            fire(jnp.minimum(c1 + 2, last), 1)
            return carry

        lax.fori_loop(0, chunks // 2, body, 0)
        drain(0)
        wait_scatter(0)
        compute(0)
        fire_scatter(last, 0)
        wait_scatter(0)
        wait_scatter(1)
        drain(1)

        plsc.subcore_barrier()
        pltpu.sync_copy(acc_s.at[pl.ds(row0, rpt)],
                        acc_out.at[cid, pl.ds(row0, rpt)])
        pltpu.sync_copy(den_s.at[pl.ds(row0, rpt)],
                        den_out.at[cid, pl.ds(row0, rpt)])

    return edge_pass


_BLK = 1000
_GRID = N // _BLK


def _tc1_body(x_ref, w_ref, a_ref, aw_ref, h_ref, al_ref, ald_ref):
    h = jnp.dot(x_ref[...], w_ref[...], preferred_element_type=F32)
    h_ref[...] = h
    al_ref[...] = jnp.dot(h, a_ref[...], preferred_element_type=F32)
    ald_ref[...] = jnp.dot(h, aw_ref[...], preferred_element_type=F32)


def _tc2_body(acc_ref, den_ref, b_ref, w_ref, a_ref, aw_ref, p_ref,
              h_ref, al_ref, ald_ref):
    acc = acc_ref[0] + acc_ref[1]
    den = jnp.dot(den_ref[0] + den_ref[1], p_ref[...],
                  preferred_element_type=F32)
    x1 = acc / jnp.maximum(den, 1e-16) + b_ref[...]
    act = jnp.where(x1 > 0, x1, jnp.exp(x1) - 1.0)
    h2 = jnp.dot(act, w_ref[...], preferred_element_type=F32)
    h_ref[...] = h2
    al_ref[...] = jnp.dot(h2, a_ref[...], preferred_element_type=F32)
    ald_ref[...] = jnp.dot(h2, aw_ref[...], preferred_element_type=F32)


def _tc3_body(acc_ref, den_ref, b_ref, p_ref, out_ref):
    acc = acc_ref[0] + acc_ref[1]
    den = jnp.dot(den_ref[0] + den_ref[1], p_ref[...],
                  preferred_element_type=F32)
    out_ref[...] = acc / jnp.maximum(den, 1e-16) + b_ref[...]


def _row_spec(cols):
    return pl.BlockSpec((_BLK, cols), lambda i: (i, 0))


def _pair_spec(cols):
    return pl.BlockSpec((2, _BLK, cols), lambda i: (0, i, 0))


def _full_spec(rows, cols):
    return pl.BlockSpec((rows, cols), lambda i: (0, 0))


def kernel(x, edge_index, W1, a1_src, a1_dst, b1, W2, a2_src, a2_dst, b2):
    epw = E // NW
    src80 = edge_index[0].reshape(NW, epw // 80, 80)
    dst80 = edge_index[1].reshape(NW, epw // 80, 80)
    src16 = edge_index[0].reshape(NW, epw // 16, 16)
    dst16 = edge_index[1].reshape(NW, epw // 16, 16)

    def pack_a(a_s, a_d, din):
        h_, c_ = a_s.shape
        cols = []
        for k in range(16):
            if k < h_:
                col = jnp.zeros((din,), F32).at[k * c_:(k + 1) * c_].set(a_s[k])
            elif 8 <= k < 8 + h_:
                hh = k - 8
                col = jnp.zeros((din,), F32).at[hh * c_:(hh + 1) * c_].set(a_d[hh])
            else:
                col = jnp.zeros((din,), F32)
            cols.append(col)
        return jnp.stack(cols, axis=1)

    A1 = pack_a(a1_src, a1_dst, 64)
    A1w = pack_a(a1_dst, a1_src, 64)
    A2 = pack_a(a2_src, a2_dst, 128)
    A2w = pack_a(a2_dst, a2_src, 128)
    P1 = jnp.concatenate(
        [jnp.kron(jnp.eye(8, dtype=F32), jnp.ones((1, 8), F32)),
         jnp.zeros((8, 64), F32)], axis=0)
    P2 = jnp.concatenate(
        [jnp.ones((1, 128), F32), jnp.zeros((15, 128), F32)], axis=0)

    rpt = NPAD // NS
    z64 = jnp.zeros((rpt, 64), F32)
    z128 = jnp.zeros((rpt, 128), F32)
    z16 = jnp.zeros((rpt, 16), F32)

    h1, al1, al1d = pl.pallas_call(
        _tc1_body,
        grid=(_GRID,),
        in_specs=[_row_spec(128), _full_spec(128, 64), _full_spec(64, 16),
                  _full_spec(64, 16)],
        out_specs=[_row_spec(64), _row_spec(16), _row_spec(16)],
        out_shape=[jax.ShapeDtypeStruct((N, 64), F32),
                   jax.ShapeDtypeStruct((N, 16), F32),
                   jax.ShapeDtypeStruct((N, 16), F32)],
    )(x, W1, A1, A1w)

    acc1, den1 = _make_edge_pass(64, 8, 8, 80)(al1, al1d, h1, src80, dst80,
                                               z64, z16)

    h2, al2, al2d = pl.pallas_call(
        _tc2_body,
        grid=(_GRID,),
        in_specs=[_pair_spec(64), _pair_spec(16),
                  _full_spec(1, 64), _full_spec(64, 128), _full_spec(128, 16),
                  _full_spec(128, 16), _full_spec(16, 64)],
        out_specs=[_row_spec(128), _row_spec(16), _row_spec(16)],
        out_shape=[jax.ShapeDtypeStruct((N, 128), F32),
                   jax.ShapeDtypeStruct((N, 16), F32),
                   jax.ShapeDtypeStruct((N, 16), F32)],
    )(acc1, den1, b1.reshape(1, 64), W2, A2, A2w, P1)

    acc2, den2 = _make_edge_pass(128, 1, 128, 16)(al2, al2d, h2, src16, dst16,
                                                  z128, z16)

    out = pl.pallas_call(
        _tc3_body,
        grid=(_GRID,),
        in_specs=[_pair_spec(128), _pair_spec(16),
                  _full_spec(1, 128), _full_spec(16, 128)],
        out_specs=_row_spec(128),
        out_shape=jax.ShapeDtypeStruct((N, 128), F32),
    )(acc2, den2, b2.reshape(1, 128), P2)

    return out
